# baseline (speedup 1.0000x reference)
import jax
import jax.numpy as jnp
from jax.experimental import pallas as pl
from jax.experimental.pallas import tpu as pltpu

_TOKEN_DIM = 1024
_CODE_DIM = 64
_HALF = (8 - 1) / 2.0
_ROWS = [512] * 16
_OFFS = [sum(_ROWS[:i]) for i in range(len(_ROWS))]
_NC = len(_ROWS)
_CH = max(_ROWS)
_NBI = 8
_NT = 8192


def _fsq_body(x_hbm, win_ref, bin_ref, wout_ref, bout_ref,
              out_hbm, loss_ref, xbufs, obufs, in_sems, out_sems):
    win = win_ref[...]
    wout = wout_ref[...]
    bin_v = bin_ref[...]
    bout_v = bout_ref[...]

    def in_copy(i, b):
        return pltpu.make_async_copy(
            x_hbm.at[pl.ds(_OFFS[i], _ROWS[i])],
            xbufs.at[b, pl.ds(0, _ROWS[i])],
            in_sems.at[b],
        )

    def out_copy(i):
        return pltpu.make_async_copy(
            obufs.at[pl.ds(_OFFS[i], _ROWS[i])],
            out_hbm.at[pl.ds(_OFFS[i], _ROWS[i])],
            out_sems.at[i],
        )

    for j in range(min(_NBI, _NC)):
        in_copy(j, j).start()

    loss = jnp.zeros((), jnp.float32)
    for i in range(_NC):
        b = i % _NBI
        in_copy(i, b).wait()

        z = jnp.dot(xbufs[b, : _ROWS[i]], win,
                    preferred_element_type=jnp.float32)

        if i + _NBI < _NC:
            in_copy(i + _NBI, b).start()

        z = z + bin_v
        z_b = jnp.tanh(z) * _HALF
        z_q = jnp.round(z_b)
        d = z_q - z_b
        loss = loss + jnp.sum(d * d)

        obufs[_OFFS[i] : _OFFS[i] + _ROWS[i]] = (
            jnp.dot(z_q * (1.0 / _HALF), wout,
                    preferred_element_type=jnp.float32)
            + bout_v
        )
        out_copy(i).start()

    loss_ref[0, 0] = loss

    for i in range(_NC):
        out_copy(i).wait()


def kernel(inputs_embeds, W_in, b_in, W_out, b_out):
    b, s, dm = inputs_embeds.shape
    n_tok = b * s
    x = inputs_embeds.reshape(n_tok, dm)

    out, loss_sum = pl.pallas_call(
        _fsq_body,
        in_specs=[
            pl.BlockSpec(memory_space=pl.ANY),
            pl.BlockSpec(memory_space=pltpu.VMEM),
            pl.BlockSpec(memory_space=pltpu.VMEM),
            pl.BlockSpec(memory_space=pltpu.VMEM),
            pl.BlockSpec(memory_space=pltpu.VMEM),
        ],
        out_specs=[
            pl.BlockSpec(memory_space=pl.ANY),
            pl.BlockSpec(memory_space=pltpu.SMEM),
        ],
        out_shape=[
            jax.ShapeDtypeStruct((n_tok, dm), jnp.float32),
            jax.ShapeDtypeStruct((1, 1), jnp.float32),
        ],
        scratch_shapes=[
            pltpu.VMEM((_NBI, _CH, dm), jnp.float32),
            pltpu.VMEM((_NT, dm), jnp.float32),
            pltpu.SemaphoreType.DMA((_NBI,)),
            pltpu.SemaphoreType.DMA((_NC,)),
        ],
    )(x, W_in, b_in.reshape(1, _CODE_DIM), W_out, b_out.reshape(1, dm))

    vq_loss = (0.35 / (n_tok * _CODE_DIM)) * loss_sum[0, 0]
    return (out.reshape(b, s, dm), vq_loss)


# speedup vs baseline: 1.0615x; 1.0615x over previous
import jax
import jax.numpy as jnp
from jax.experimental import pallas as pl
from jax.experimental.pallas import tpu as pltpu

_TOKEN_DIM = 1024
_CODE_DIM = 64
_HALF = (8 - 1) / 2.0
_ROWS = [512] + [1024] * 7 + [512]
_OFFS = [sum(_ROWS[:i]) for i in range(len(_ROWS))]
_NC = len(_ROWS)
_CH = max(_ROWS)
_NBI = 6
_NT = 8192


def _fsq_body(x_hbm, win_ref, bin_ref, wout_ref, bout_ref,
              out_hbm, loss_ref, xbufs, obufs, in_sems, out_sems):
    win = win_ref[...]
    wout = wout_ref[...]
    bin_v = bin_ref[...]
    bout_v = bout_ref[...]

    def in_copy(i, b):
        return pltpu.make_async_copy(
            x_hbm.at[pl.ds(_OFFS[i], _ROWS[i])],
            xbufs.at[b, pl.ds(0, _ROWS[i])],
            in_sems.at[b],
        )

    def out_copy(i):
        return pltpu.make_async_copy(
            obufs.at[pl.ds(_OFFS[i], _ROWS[i])],
            out_hbm.at[pl.ds(_OFFS[i], _ROWS[i])],
            out_sems.at[i],
        )

    for j in range(min(_NBI, _NC)):
        in_copy(j, j).start()

    loss = jnp.zeros((), jnp.float32)
    for i in range(_NC):
        b = i % _NBI
        in_copy(i, b).wait()

        z = jnp.dot(xbufs[b, : _ROWS[i]], win,
                    preferred_element_type=jnp.float32)

        if i + _NBI < _NC:
            in_copy(i + _NBI, b).start()

        z = z + bin_v
        z_b = jnp.tanh(z) * _HALF
        z_q = jnp.round(z_b)
        d = z_q - z_b
        loss = loss + jnp.sum(d * d)

        obufs[_OFFS[i] : _OFFS[i] + _ROWS[i]] = (
            jnp.dot(z_q * (1.0 / _HALF), wout,
                    preferred_element_type=jnp.float32)
            + bout_v
        )
        out_copy(i).start()

    loss_ref[0, 0] = loss

    for i in range(_NC):
        out_copy(i).wait()


def kernel(inputs_embeds, W_in, b_in, W_out, b_out):
    b, s, dm = inputs_embeds.shape
    n_tok = b * s
    x = inputs_embeds.reshape(n_tok, dm)

    out, loss_sum = pl.pallas_call(
        _fsq_body,
        in_specs=[
            pl.BlockSpec(memory_space=pl.ANY),
            pl.BlockSpec(memory_space=pltpu.VMEM),
            pl.BlockSpec(memory_space=pltpu.VMEM),
            pl.BlockSpec(memory_space=pltpu.VMEM),
            pl.BlockSpec(memory_space=pltpu.VMEM),
        ],
        out_specs=[
            pl.BlockSpec(memory_space=pl.ANY),
            pl.BlockSpec(memory_space=pltpu.SMEM),
        ],
        out_shape=[
            jax.ShapeDtypeStruct((n_tok, dm), jnp.float32),
            jax.ShapeDtypeStruct((1, 1), jnp.float32),
        ],
        scratch_shapes=[
            pltpu.VMEM((_NBI, _CH, dm), jnp.float32),
            pltpu.VMEM((_NT, dm), jnp.float32),
            pltpu.SemaphoreType.DMA((_NBI,)),
            pltpu.SemaphoreType.DMA((_NC,)),
        ],
    )(x, W_in, b_in.reshape(1, _CODE_DIM), W_out, b_out.reshape(1, dm))

    vq_loss = (0.35 / (n_tok * _CODE_DIM)) * loss_sum[0, 0]
    return (out.reshape(b, s, dm), vq_loss)
